# baseline (speedup 1.0000x reference)
import functools

import jax
import jax.numpy as jnp
from jax import lax
from jax.experimental import pallas as pl
from jax.experimental.pallas import tpu as pltpu
from jax.experimental.pallas import tpu_sc as plsc

_K = 256
_B = 128
_N = 32768
_NC = 2
_NS = 16
_NW = _NC * _NS
_RPW = _B // _NW
_L = 16
_NV = _N // _L
_NQ = 8
_QV = _NV // _NQ
_CAP = 512
_RS = _CAP + 32
_CIDX_SZ = (_NQ - 1) * _RS + _QV * _L + _L
_GCAP = _NQ * _CAP + 80
_SS = 32
_UNROLL = 8


def _u32_to_f32_vec(mid_u32_scalar):
  mid = jnp.full((_L,), mid_u32_scalar, dtype=jnp.uint32)
  neg = mid < jnp.uint32(0x80000000)
  bits = jnp.where(neg, ~mid, mid ^ jnp.uint32(0x80000000))
  return plsc.bitcast(bits, jnp.float32)


def _f32_to_u32(v):
  bu = plsc.bitcast(v, jnp.uint32)
  neg = bu >= jnp.uint32(0x80000000)
  return jnp.where(neg, ~bu, bu ^ jnp.uint32(0x80000000))


def _count_ge(row_v, thr_f):
  def body(i, acc):
    for j in range(_UNROLL):
      v = row_v[pl.ds((i * _UNROLL + j) * _L, _L)]
      acc = acc + jnp.where(v >= thr_f, jnp.int32(1), jnp.int32(0))
    return acc
  acc = lax.fori_loop(0, _NV // _UNROLL, body,
                      jnp.zeros((_L,), jnp.int32))
  return jnp.sum(acc)


def kernel(x):
  mesh = plsc.VectorSubcoreMesh(
      core_axis_name="c", subcore_axis_name="s",
      num_cores=_NC, num_subcores=_NS)

  @functools.partial(
      pl.kernel,
      out_type=jax.ShapeDtypeStruct((_B, _N), jnp.float32),
      mesh=mesh,
      scratch_types=[
          pltpu.VMEM((_N,), jnp.float32),
          pltpu.VMEM((_N,), jnp.float32),
          pltpu.VMEM((_N,), jnp.float32),
          pltpu.VMEM((_CIDX_SZ,), jnp.int32),
          pltpu.VMEM((_GCAP,), jnp.float32),
          pltpu.VMEM((_GCAP,), jnp.int32),
          pltpu.VMEM((_K + _L,), jnp.int32),
          pltpu.SemaphoreType.DMA,
          pltpu.SemaphoreType.DMA,
          pltpu.SemaphoreType.DMA,
      ],
      compiler_params=pltpu.CompilerParams(needs_layout_passes=False),
  )
  def _topk_mask(x_hbm, out_hbm, rowa_v, rowb_v, zero_v, cidx_v, gval_v,
                 gidx_v, kept_v, isem_a, isem_b, osem):
    wid = lax.axis_index("s") * _NC + lax.axis_index("c")
    iota = lax.iota(jnp.int32, _L)
    zero_f = jnp.zeros((_L,), jnp.float32)
    nan_f = jnp.full((_L,), jnp.float32(jnp.nan))
    true_m = iota < jnp.int32(_L)

    def zb(i, _):
      for j in range(_UNROLL):
        zero_v[pl.ds((i * _UNROLL + j) * _L, _L)] = zero_f
      return _
    lax.fori_loop(0, _NV // _UNROLL, zb, jnp.int32(0))

    def do_row(r, row_v, h_out_prev, tlow_in):
      row = wid * _RPW + r

      if tlow_in is None:
        def stats(i, c):
          s, q = c
          for j in range(4):
            v = row_v[pl.ds(((i * 4 + j) * _SS) * _L, _L)]
            s = s + v
            q = q + v * v
          return (s, q)
        s_v, q_v = lax.fori_loop(
            0, _NV // _SS // 4, stats, (zero_f, zero_f))
        inv_n = jnp.float32(1.0 / ((_NV // _SS) * _L))
        mean_s = jnp.sum(s_v) * inv_n
        var_s = jnp.maximum(jnp.sum(q_v) * inv_n - mean_s * mean_s,
                            jnp.float32(1e-30))
        var_v = jnp.full((_L,), var_s)
        vb = plsc.bitcast(var_v, jnp.int32)
        y = plsc.bitcast(jnp.int32(0x5F3759DF) - (vb >> 1), jnp.float32)
        half = jnp.float32(0.5) * var_v
        y = y * (jnp.float32(1.5) - half * y * y)
        y = y * (jnp.float32(1.5) - half * y * y)
        tlow = jnp.full((_L,), mean_s) + jnp.float32(2.1) * var_v * y
      else:
        tlow = tlow_in

      v_cur = [row_v[pl.ds((c * _QV) * _L, _L)] for c in range(_NQ)]

      def step(i, vs, ptrs, mx, lookahead):
        new_vs, new_ptrs = [], []
        for c in range(_NQ):
          off = (c * _QV + i) * _L
          v = vs[c]
          m = v >= tlow
          mx = jnp.maximum(mx, v)
          plsc.store_compressed(
              cidx_v.at[pl.ds(c * _RS + ptrs[c], _L)], iota + off, mask=m)
          new_ptrs.append(
              ptrs[c] + plsc.all_reduce_population_count(m)[0])
          if lookahead:
            new_vs.append(row_v[pl.ds(off + _L, _L)])
        return new_vs, new_ptrs, mx

      def fused(i, carry):
        vs, ptrs, mx = carry[:_NQ], carry[_NQ:2 * _NQ], carry[2 * _NQ]
        vs, ptrs, mx = step(i, list(vs), list(ptrs), mx, True)
        return (*vs, *ptrs, mx)

      init = (*v_cur, *((jnp.int32(0),) * _NQ),
              jnp.full((_L,), -jnp.inf, jnp.float32))
      carry = lax.fori_loop(0, _QV - 1, fused, init)
      _, ptrs, mx_v = (carry[:_NQ], carry[_NQ:2 * _NQ], carry[2 * _NQ])
      _, ptrs, mx_v = step(_QV - 1, list(carry[:_NQ]), list(ptrs), mx_v,
                           False)

      n_c = ptrs[0]
      for c in range(1, _NQ):
        n_c = n_c + ptrs[c]
      ok = n_c >= jnp.int32(_K)
      for c in range(_NQ):
        ok = ok & (ptrs[c] <= jnp.int32(_CAP))

      if h_out_prev is not None:
        h_out_prev.wait()

      return pltpu.async_copy(zero_v, out_hbm.at[row], osem), tlow

    bufs = (rowa_v, rowb_v)
    isems = (isem_a, isem_b)
    base = wid * _RPW
    h_in = pltpu.async_copy(x_hbm.at[base], bufs[0], isems[0])
    h_out, tlow = None, None
    for r in range(_RPW):
      h_in.wait()
      if r + 1 < _RPW:
        h_in = pltpu.async_copy(
            x_hbm.at[base + r + 1], bufs[(r + 1) % 2], isems[(r + 1) % 2])
      h_out, tlow = do_row(r, bufs[r % 2], h_out, tlow)
    h_out.wait()

  return _topk_mask(x)
